# baseline (speedup 1.0000x reference)
import functools

import numpy as np
import jax
import jax.numpy as jnp
from jax import lax
from jax.experimental import pallas as pl
from jax.experimental.pallas import tpu as pltpu
from jax.experimental.pallas import tpu_sc as plsc

N = 10000
E = 160000
EMB = 16
RAD = 32
CUTOFF = 5.0

NC = 2
NS = 16
NW = NC * NS
EPER = E // NW
GROUPS = EPER // 16
EPAD_T = (GROUPS + 1) * 16
NPAD = 10240
CH = 8000
NCHUNK = E // CH

_MEANS = np.linspace(np.exp(-CUTOFF), 1.0, RAD).astype(np.float32)
_BETA = float((2.0 / RAD * (1.0 - np.exp(-CUTOFF))) ** -2)


def _sc_gather_body(px_h, py_h, pz_h, ctr_h, nbr_h, q_h,
                    px_v, py_v, pz_v, ctr_v, nbr_v, q_v):
    cid = lax.axis_index("c")
    sid = lax.axis_index("s")
    wid = sid * NC + cid
    base = wid * EPER
    z16 = jnp.zeros((16,), jnp.int32)
    ctr_v[pl.ds(EPAD_T - 16, 16)] = z16
    nbr_v[pl.ds(EPAD_T - 16, 16)] = z16
    pltpu.sync_copy(px_h, px_v)
    pltpu.sync_copy(py_h, py_v)
    pltpu.sync_copy(pz_h, pz_v)
    pltpu.sync_copy(ctr_h.at[pl.ds(base, EPER)], ctr_v.at[pl.ds(0, EPER)])
    pltpu.sync_copy(nbr_h.at[pl.ds(base, EPER)], nbr_v.at[pl.ds(0, EPER)])

    def body(i, carry):
        ic = ctr_v[pl.ds(i * 16, 16)]
        inb = nbr_v[pl.ds(i * 16, 16)]
        dx = plsc.load_gather(px_v, [inb]) - plsc.load_gather(px_v, [ic])
        dy = plsc.load_gather(py_v, [inb]) - plsc.load_gather(py_v, [ic])
        dz = plsc.load_gather(pz_v, [inb]) - plsc.load_gather(pz_v, [ic])
        q_v[pl.ds(i * 16, 16)] = dx * dx + dy * dy + dz * dz
        return carry

    lax.fori_loop(0, GROUPS + 1, body, 0)
    pltpu.sync_copy(q_v.at[pl.ds(0, EPER)], q_h.at[pl.ds(base, EPER)])


UNR = 10
_M0 = float(_MEANS[0])
_DM = float((1.0 - np.exp(-CUTOFF)) / (RAD - 1))
_SQB = float(np.sqrt(_BETA * np.log2(np.e)))
_C = (0.9999998785131207, -0.6931420291420406, 0.24017432594448082,
      -0.055291340610177604, 0.009206239823170195, -0.0009471897117002266)


def _exp2neg(t):
    tc = jnp.minimum(t, 126.0)
    i = tc.astype(jnp.int32)
    f = tc - i.astype(jnp.float32)
    p = _C[0] + f * (_C[1] + f * (_C[2] + f * (_C[3] + f * (_C[4] + f * _C[5]))))
    s = plsc.bitcast((127 - i) << 23, jnp.float32)
    return p * s


def _sc_scatter_body(er_h, env_h, nbr_h, gm_h, g32_h,
                     idx_a, era_v, eva_v, idx_b, erb_v, evb_v,
                     ei_v, ev_v, g_v, g2_v, g32_v,
                     sem_ia, sem_ea, sem_va, sem_ib, sem_eb, sem_vb):
    cid = lax.axis_index("c")
    sid = lax.axis_index("s")
    wid = sid * NC + cid
    mean_t = (_M0 + lax.convert_element_type(wid, jnp.float32) * _DM) * _SQB

    zf = jnp.zeros((16,), jnp.float32)

    def zbody(i, carry):
        for u in range(8):
            g_v[pl.ds(i * 128 + u * 16, 16)] = zf
            g2_v[pl.ds(i * 128 + u * 16, 16)] = zf
            g32_v[pl.ds(i * 128 + u * 16, 16)] = zf
        return carry

    lax.fori_loop(0, NPAD // 128, zbody, 0)

    bufs = ((idx_a, era_v, eva_v, sem_ia, sem_ea, sem_va),
            (idx_b, erb_v, evb_v, sem_ib, sem_eb, sem_vb))

    def start(c, bi):
        ia, ea, va, si, se, sv = bufs[bi]
        d1 = pltpu.async_copy(nbr_h.at[pl.ds(c * CH, CH)], ia, si)
        d2 = pltpu.async_copy(er_h.at[pl.ds(c * CH, CH)], ea, se)
        d3 = pltpu.async_copy(env_h.at[pl.ds(c * CH, CH)], va, sv)
        return d1, d2, d3

    pend = start(0, 0)
    for c in range(NCHUNK):
        bi = c & 1
        ia, ea, va, _, _, _ = bufs[bi]
        nxt = start(c + 1, 1 - bi) if c + 1 < NCHUNK else None
        pend[0].wait()
        pend[1].wait()
        pend[2].wait()

        def sbody(g, carry):
            for u in range(UNR):
                off = g * (16 * UNR) + u * 16
                iv = ia[pl.ds(off, 16)]
                d = ea[pl.ds(off, 16)] - mean_t
                vv = _exp2neg(d * d) * va[pl.ds(off, 16)]
                plsc.addupdate_scatter(g_v if u % 2 == 0 else g2_v, [iv], vv)
            return carry

        lax.fori_loop(0, CH // (16 * UNR), sbody, 0)
        pend = nxt

    def mbody(i, carry):
        for u in range(8):
            off = i * 128 + u * 16
            g_v[pl.ds(off, 16)] = g_v[pl.ds(off, 16)] + g2_v[pl.ds(off, 16)]
        return carry

    lax.fori_loop(0, NPAD // 128, mbody, 0)

    ebase = wid * EPER
    z16 = jnp.zeros((16,), jnp.int32)
    ei_v[pl.ds(EPAD_T - 16, 16)] = z16
    ev_v[pl.ds(EPAD_T - 16, 16)] = jnp.zeros((16,), jnp.float32)
    pltpu.sync_copy(nbr_h.at[pl.ds(ebase, EPER)], ei_v.at[pl.ds(0, EPER)])
    pltpu.sync_copy(env_h.at[pl.ds(ebase, EPER)], ev_v.at[pl.ds(0, EPER)])

    def ebody(g, carry):
        iv = ei_v[pl.ds(g * 16, 16)]
        vv = ev_v[pl.ds(g * 16, 16)]
        plsc.addupdate_scatter(g32_v, [iv], vv)
        return carry

    lax.fori_loop(0, GROUPS + 1, ebody, 0)

    pltpu.sync_copy(g_v, gm_h.at[pl.ds(wid * NPAD, NPAD)])
    pltpu.sync_copy(g32_v, g32_h.at[pl.ds(wid * NPAD, NPAD)])


@functools.cache
def _sc_kernels():
    mesh = plsc.VectorSubcoreMesh(
        core_axis_name="c", subcore_axis_name="s",
        num_cores=NC, num_subcores=NS)
    params = pltpu.CompilerParams(needs_layout_passes=False)
    sc_gather = pl.kernel(
        _sc_gather_body,
        out_type=jax.ShapeDtypeStruct((E,), jnp.float32),
        mesh=mesh,
        compiler_params=params,
        scratch_types=[
            pltpu.VMEM((N,), jnp.float32),
            pltpu.VMEM((N,), jnp.float32),
            pltpu.VMEM((N,), jnp.float32),
            pltpu.VMEM((EPAD_T,), jnp.int32),
            pltpu.VMEM((EPAD_T,), jnp.int32),
            pltpu.VMEM((EPAD_T,), jnp.float32),
        ],
    )
    sc_scatter = pl.kernel(
        _sc_scatter_body,
        out_type=(jax.ShapeDtypeStruct((NW * NPAD,), jnp.float32),
                  jax.ShapeDtypeStruct((NW * NPAD,), jnp.float32)),
        mesh=mesh,
        compiler_params=params,
        scratch_types=[
            pltpu.VMEM((CH,), jnp.int32),
            pltpu.VMEM((CH,), jnp.float32),
            pltpu.VMEM((CH,), jnp.float32),
            pltpu.VMEM((CH,), jnp.int32),
            pltpu.VMEM((CH,), jnp.float32),
            pltpu.VMEM((CH,), jnp.float32),
            pltpu.VMEM((EPAD_T,), jnp.int32),
            pltpu.VMEM((EPAD_T,), jnp.float32),
            pltpu.VMEM((NPAD,), jnp.float32),
            pltpu.VMEM((NPAD,), jnp.float32),
            pltpu.VMEM((NPAD,), jnp.float32),
            pltpu.SemaphoreType.DMA,
            pltpu.SemaphoreType.DMA,
            pltpu.SemaphoreType.DMA,
            pltpu.SemaphoreType.DMA,
            pltpu.SemaphoreType.DMA,
            pltpu.SemaphoreType.DMA,
        ],
    )
    return sc_gather, sc_scatter


EPAD_E = 163840
BE = 16384


def _rbf1_body(q_ref, er_ref, env_ref):
    q = q_ref[...]
    r = jnp.sqrt(q + 1e-12)
    er_ref[...] = jnp.exp(-r) * np.float32(_SQB)
    env_ref[...] = jnp.where(
        r < CUTOFF, 0.5 * (jnp.cos(np.float32(np.pi) / CUTOFF * r) + 1.0), 0.0)


_rbf1 = pl.pallas_call(
    _rbf1_body,
    grid=(EPAD_E // BE,),
    in_specs=[pl.BlockSpec((BE,), lambda i: (i,))],
    out_specs=(pl.BlockSpec((BE,), lambda i: (i,)),
               pl.BlockSpec((BE,), lambda i: (i,))),
    out_shape=(jax.ShapeDtypeStruct((EPAD_E,), jnp.float32),
               jax.ShapeDtypeStruct((EPAD_E,), jnp.float32)),
)


BN = 1024


def _dense_body(xt_ref, gm_ref, g32_ref, wpre_ref, wpost_ref, wlin_ref,
                b_ref, o_ref):
    def lin(Wm, v):
        return jnp.dot(Wm, v, preferred_element_type=jnp.float32)

    x = [xt_ref[k] for k in range(9)]
    fro2 = x[0] * x[0]
    for k in range(1, 9):
        fro2 = fro2 + x[k] * x[k]
    inv = 1.0 / (fro2 + 1.0)
    xn = [x[k] * inv for k in range(9)]
    m = (xn[0] + xn[4] + xn[8]) / 3.0

    P, Q2, D_, T = (wpre_ref[0], wpre_ref[1], wpre_ref[2], wpre_ref[3])
    Tm = lin(T, m)
    Y = [None] * 9
    for i in range(3):
        for j in range(3):
            k, kt = 3 * i + j, 3 * j + i
            if i == j:
                Y[k] = lin(D_, xn[k]) + Tm
            else:
                Y[k] = lin(P, xn[k]) + lin(Q2, xn[kt])

    genv = jnp.sum(g32_ref[...], axis=0, keepdims=True)
    C = lin(wlin_ref[...], gm_ref[...]) + b_ref[...] * genv
    cI, cA, cS = C[0:EMB], C[EMB:2 * EMB], C[2 * EMB:3 * EMB]
    M = [None] * 9
    for i in range(3):
        for j in range(3):
            k, kt = 3 * i + j, 3 * j + i
            if i == j:
                M[k] = cI * m + cS * (xn[k] - m)
            else:
                M[k] = cA * (0.5 * (xn[k] - xn[kt])) + cS * (0.5 * (xn[k] + xn[kt]))

    Z = [None] * 9
    for i in range(3):
        for j in range(3):
            acc = None
            for t in range(3):
                term = Y[3 * i + t] * M[3 * t + j] + M[3 * i + t] * Y[3 * t + j]
                acc = term if acc is None else acc + term
            Z[3 * i + j] = acc

    n2 = None
    for k in range(9):
        zk1 = Z[k] + 1.0
        n2 = zk1 * zk1 if n2 is None else n2 + zk1 * zk1
    zn = [Z[k] / n2 for k in range(9)]
    m2 = (zn[0] + zn[4] + zn[8]) / 3.0

    Pp, Qp, Dp, Tp = (wpost_ref[0], wpost_ref[1], wpost_ref[2], wpost_ref[3])
    T2 = lin(Tp, m2)
    Y2 = [None] * 9
    for i in range(3):
        for j in range(3):
            k, kt = 3 * i + j, 3 * j + i
            if i == j:
                Y2[k] = lin(Dp, zn[k]) + T2
            else:
                Y2[k] = lin(Pp, zn[k]) + lin(Qp, zn[kt])

    for i in range(3):
        for j in range(3):
            acc = Y2[3 * i + j]
            for t in range(3):
                acc = acc + Y2[3 * i + t] * Y2[3 * t + j]
            o_ref[3 * i + j] = acc


_dense = pl.pallas_call(
    _dense_body,
    grid=(NPAD // BN,),
    in_specs=[
        pl.BlockSpec((9, EMB, BN), lambda i: (0, 0, i)),
        pl.BlockSpec((RAD, BN), lambda i: (0, i)),
        pl.BlockSpec((NW, BN), lambda i: (0, i)),
        pl.BlockSpec((4, EMB, EMB), lambda i: (0, 0, 0)),
        pl.BlockSpec((4, EMB, EMB), lambda i: (0, 0, 0)),
        pl.BlockSpec((3 * EMB, RAD), lambda i: (0, 0)),
        pl.BlockSpec((3 * EMB, 1), lambda i: (0, 0)),
    ],
    out_specs=pl.BlockSpec((9, EMB, BN), lambda i: (0, 0, i)),
    out_shape=jax.ShapeDtypeStruct((9, EMB, NPAD), jnp.float32),
)


def kernel(X, neighbour_index, positions, W_I_pre, W_A_pre, W_S_pre,
           W_I_post, W_A_post, W_S_post, lin_W, lin_b):
    ctr = neighbour_index[0]
    nbr = neighbour_index[1]
    px = jnp.asarray(positions[:, 0])
    py = jnp.asarray(positions[:, 1])
    pz = jnp.asarray(positions[:, 2])

    sc_gather, sc_scatter = _sc_kernels()
    q = sc_gather(px, py, pz, ctr, nbr)
    qp = jnp.pad(q, (0, EPAD_E - E))
    er, env = _rbf1(qp)
    gm_flat, g32_flat = sc_scatter(er, env, nbr)
    gm = gm_flat.reshape(NW, NPAD)[:RAD]
    g32 = g32_flat.reshape(NW, NPAD)

    xt = jnp.pad(X.transpose(2, 3, 1, 0).reshape(9, EMB, N),
                 ((0, 0), (0, 0), (0, NPAD - N)))

    wpre = jnp.stack([0.5 * (W_A_pre + W_S_pre), 0.5 * (W_S_pre - W_A_pre),
                      W_S_pre, W_I_pre - W_S_pre])
    wpost = jnp.stack([0.5 * (W_A_post + W_S_post), 0.5 * (W_S_post - W_A_post),
                       W_S_post, W_I_post - W_S_post])

    out_t = _dense(xt, gm, g32, wpre, wpost, lin_W, lin_b[:, None])
    return out_t[:, :, :N].reshape(3, 3, EMB, N).transpose(3, 2, 0, 1)


# speedup vs baseline: 2.0038x; 2.0038x over previous
import functools

import numpy as np
import jax
import jax.numpy as jnp
from jax import lax
from jax.experimental import pallas as pl
from jax.experimental.pallas import tpu as pltpu
from jax.experimental.pallas import tpu_sc as plsc

N = 10000
E = 160000
EMB = 16
RAD = 32
CUTOFF = 5.0

NC = 2
NS = 16
NW = NC * NS
EPER = E // NW
GROUPS = EPER // 16
EPAD_T = (GROUPS + 1) * 16
NPAD = 10240
CH = 8000
NCHUNK = E // CH

_MEANS = np.linspace(np.exp(-CUTOFF), 1.0, RAD).astype(np.float32)
_BETA = float((2.0 / RAD * (1.0 - np.exp(-CUTOFF))) ** -2)


def _sc_gather_body(px_h, py_h, pz_h, ctr_h, nbr_h, q_h,
                    px_v, py_v, pz_v, ctr_v, nbr_v, q_v):
    cid = lax.axis_index("c")
    sid = lax.axis_index("s")
    wid = sid * NC + cid
    base = wid * EPER
    z16 = jnp.zeros((16,), jnp.int32)
    ctr_v[pl.ds(EPAD_T - 16, 16)] = z16
    nbr_v[pl.ds(EPAD_T - 16, 16)] = z16
    pltpu.sync_copy(px_h, px_v)
    pltpu.sync_copy(py_h, py_v)
    pltpu.sync_copy(pz_h, pz_v)
    pltpu.sync_copy(ctr_h.at[pl.ds(base, EPER)], ctr_v.at[pl.ds(0, EPER)])
    pltpu.sync_copy(nbr_h.at[pl.ds(base, EPER)], nbr_v.at[pl.ds(0, EPER)])

    def body(i, carry):
        ic = ctr_v[pl.ds(i * 16, 16)]
        inb = nbr_v[pl.ds(i * 16, 16)]
        dx = plsc.load_gather(px_v, [inb]) - plsc.load_gather(px_v, [ic])
        dy = plsc.load_gather(py_v, [inb]) - plsc.load_gather(py_v, [ic])
        dz = plsc.load_gather(pz_v, [inb]) - plsc.load_gather(pz_v, [ic])
        q_v[pl.ds(i * 16, 16)] = dx * dx + dy * dy + dz * dz
        return carry

    lax.fori_loop(0, GROUPS + 1, body, 0)
    pltpu.sync_copy(q_v.at[pl.ds(0, EPER)], q_h.at[pl.ds(base, EPER)])


UNR = 10
_M0 = float(_MEANS[0])
_DM = float((1.0 - np.exp(-CUTOFF)) / (RAD - 1))
_SQB = float(np.sqrt(_BETA))
EHALF = E // 2
NCHUNK2 = EHALF // CH


def _sc_scatter_body(er_h, env_h, nbr_h, gm_h, g32_h,
                     idx_a, era_v, eva_v, idx_b, erb_v, evb_v,
                     ei_v, ev_v, g_v, g2_v, g32_v,
                     sem_ia, sem_ea, sem_va, sem_ib, sem_eb, sem_vb):
    cid = lax.axis_index("c")
    sid = lax.axis_index("s")
    wid = sid * NC + cid
    k16 = lax.rem(wid, 16)
    half = wid // 16
    f0 = k16 * 2
    f0f = lax.convert_element_type(f0, jnp.float32)
    mean_a = (_M0 + f0f * _DM) * _SQB
    mean_b = mean_a + _DM * _SQB

    zf = jnp.zeros((16,), jnp.float32)

    def zbody(i, carry):
        for u in range(8):
            g_v[pl.ds(i * 128 + u * 16, 16)] = zf
            g2_v[pl.ds(i * 128 + u * 16, 16)] = zf
            g32_v[pl.ds(i * 128 + u * 16, 16)] = zf
        return carry

    lax.fori_loop(0, NPAD // 128, zbody, 0)

    bufs = ((idx_a, era_v, eva_v, sem_ia, sem_ea, sem_va),
            (idx_b, erb_v, evb_v, sem_ib, sem_eb, sem_vb))
    ebase0 = half * EHALF

    def start(c, bi):
        ia, ea, va, si, se, sv = bufs[bi]
        d1 = pltpu.async_copy(nbr_h.at[pl.ds(ebase0 + c * CH, CH)], ia, si)
        d2 = pltpu.async_copy(er_h.at[pl.ds(ebase0 + c * CH, CH)], ea, se)
        d3 = pltpu.async_copy(env_h.at[pl.ds(ebase0 + c * CH, CH)], va, sv)
        return d1, d2, d3

    pend = start(0, 0)
    for c in range(NCHUNK2):
        bi = c & 1
        ia, ea, va, _, _, _ = bufs[bi]
        nxt = start(c + 1, 1 - bi) if c + 1 < NCHUNK2 else None
        pend[0].wait()
        pend[1].wait()
        pend[2].wait()

        def sbody(g, carry):
            for u in range(UNR):
                off = g * (16 * UNR) + u * 16
                iv = ia[pl.ds(off, 16)]
                erv = ea[pl.ds(off, 16)]
                env = va[pl.ds(off, 16)]
                da = erv - mean_a
                db = erv - mean_b
                plsc.addupdate_scatter(g_v, [iv], jnp.exp(-(da * da)) * env)
                plsc.addupdate_scatter(g2_v, [iv], jnp.exp(-(db * db)) * env)
            return carry

        lax.fori_loop(0, CH // (16 * UNR), sbody, 0)
        pend = nxt

    ebase = wid * EPER
    z16 = jnp.zeros((16,), jnp.int32)
    ei_v[pl.ds(EPAD_T - 16, 16)] = z16
    ev_v[pl.ds(EPAD_T - 16, 16)] = jnp.zeros((16,), jnp.float32)
    pltpu.sync_copy(nbr_h.at[pl.ds(ebase, EPER)], ei_v.at[pl.ds(0, EPER)])
    pltpu.sync_copy(env_h.at[pl.ds(ebase, EPER)], ev_v.at[pl.ds(0, EPER)])

    def ebody(g, carry):
        iv = ei_v[pl.ds(g * 16, 16)]
        vv = ev_v[pl.ds(g * 16, 16)]
        plsc.addupdate_scatter(g32_v, [iv], vv)
        return carry

    lax.fori_loop(0, GROUPS + 1, ebody, 0)

    pltpu.sync_copy(g_v, gm_h.at[pl.ds((wid * 2) * NPAD, NPAD)])
    pltpu.sync_copy(g2_v, gm_h.at[pl.ds((wid * 2 + 1) * NPAD, NPAD)])
    pltpu.sync_copy(g32_v, g32_h.at[pl.ds(wid * NPAD, NPAD)])


@functools.cache
def _sc_kernels():
    mesh = plsc.VectorSubcoreMesh(
        core_axis_name="c", subcore_axis_name="s",
        num_cores=NC, num_subcores=NS)
    params = pltpu.CompilerParams(needs_layout_passes=False)
    sc_gather = pl.kernel(
        _sc_gather_body,
        out_type=jax.ShapeDtypeStruct((E,), jnp.float32),
        mesh=mesh,
        compiler_params=params,
        scratch_types=[
            pltpu.VMEM((N,), jnp.float32),
            pltpu.VMEM((N,), jnp.float32),
            pltpu.VMEM((N,), jnp.float32),
            pltpu.VMEM((EPAD_T,), jnp.int32),
            pltpu.VMEM((EPAD_T,), jnp.int32),
            pltpu.VMEM((EPAD_T,), jnp.float32),
        ],
    )
    sc_scatter = pl.kernel(
        _sc_scatter_body,
        out_type=(jax.ShapeDtypeStruct((2 * NW * NPAD,), jnp.float32),
                  jax.ShapeDtypeStruct((NW * NPAD,), jnp.float32)),
        mesh=mesh,
        compiler_params=params,
        scratch_types=[
            pltpu.VMEM((CH,), jnp.int32),
            pltpu.VMEM((CH,), jnp.float32),
            pltpu.VMEM((CH,), jnp.float32),
            pltpu.VMEM((CH,), jnp.int32),
            pltpu.VMEM((CH,), jnp.float32),
            pltpu.VMEM((CH,), jnp.float32),
            pltpu.VMEM((EPAD_T,), jnp.int32),
            pltpu.VMEM((EPAD_T,), jnp.float32),
            pltpu.VMEM((NPAD,), jnp.float32),
            pltpu.VMEM((NPAD,), jnp.float32),
            pltpu.VMEM((NPAD,), jnp.float32),
            pltpu.SemaphoreType.DMA,
            pltpu.SemaphoreType.DMA,
            pltpu.SemaphoreType.DMA,
            pltpu.SemaphoreType.DMA,
            pltpu.SemaphoreType.DMA,
            pltpu.SemaphoreType.DMA,
        ],
    )
    return sc_gather, sc_scatter


EPAD_E = 163840
BE = 16384


def _rbf1_body(q_ref, er_ref, env_ref):
    q = q_ref[...]
    r = jnp.sqrt(q + 1e-12)
    er_ref[...] = jnp.exp(-r) * np.float32(_SQB)
    env_ref[...] = jnp.where(
        r < CUTOFF, 0.5 * (jnp.cos(np.float32(np.pi) / CUTOFF * r) + 1.0), 0.0)


_rbf1 = pl.pallas_call(
    _rbf1_body,
    grid=(EPAD_E // BE,),
    in_specs=[pl.BlockSpec((BE,), lambda i: (i,))],
    out_specs=(pl.BlockSpec((BE,), lambda i: (i,)),
               pl.BlockSpec((BE,), lambda i: (i,))),
    out_shape=(jax.ShapeDtypeStruct((EPAD_E,), jnp.float32),
               jax.ShapeDtypeStruct((EPAD_E,), jnp.float32)),
)


BN = 1024


def _dense_body(xt_ref, gm_ref, g32_ref, wpre_ref, wpost_ref, wlin_ref,
                b_ref, o_ref):
    def lin(Wm, v):
        return jnp.dot(Wm, v, preferred_element_type=jnp.float32)

    x = [xt_ref[k] for k in range(9)]
    fro2 = x[0] * x[0]
    for k in range(1, 9):
        fro2 = fro2 + x[k] * x[k]
    inv = 1.0 / (fro2 + 1.0)
    xn = [x[k] * inv for k in range(9)]
    m = (xn[0] + xn[4] + xn[8]) / 3.0

    P, Q2, D_, T = (wpre_ref[0], wpre_ref[1], wpre_ref[2], wpre_ref[3])
    Tm = lin(T, m)
    Y = [None] * 9
    for i in range(3):
        for j in range(3):
            k, kt = 3 * i + j, 3 * j + i
            if i == j:
                Y[k] = lin(D_, xn[k]) + Tm
            else:
                Y[k] = lin(P, xn[k]) + lin(Q2, xn[kt])

    genv = jnp.sum(g32_ref[...], axis=0, keepdims=True)
    C = lin(wlin_ref[...], gm_ref[0] + gm_ref[1]) + b_ref[...] * genv
    cI, cA, cS = C[0:EMB], C[EMB:2 * EMB], C[2 * EMB:3 * EMB]
    M = [None] * 9
    for i in range(3):
        for j in range(3):
            k, kt = 3 * i + j, 3 * j + i
            if i == j:
                M[k] = cI * m + cS * (xn[k] - m)
            else:
                M[k] = cA * (0.5 * (xn[k] - xn[kt])) + cS * (0.5 * (xn[k] + xn[kt]))

    Z = [None] * 9
    for i in range(3):
        for j in range(3):
            acc = None
            for t in range(3):
                term = Y[3 * i + t] * M[3 * t + j] + M[3 * i + t] * Y[3 * t + j]
                acc = term if acc is None else acc + term
            Z[3 * i + j] = acc

    n2 = None
    for k in range(9):
        zk1 = Z[k] + 1.0
        n2 = zk1 * zk1 if n2 is None else n2 + zk1 * zk1
    zn = [Z[k] / n2 for k in range(9)]
    m2 = (zn[0] + zn[4] + zn[8]) / 3.0

    Pp, Qp, Dp, Tp = (wpost_ref[0], wpost_ref[1], wpost_ref[2], wpost_ref[3])
    T2 = lin(Tp, m2)
    Y2 = [None] * 9
    for i in range(3):
        for j in range(3):
            k, kt = 3 * i + j, 3 * j + i
            if i == j:
                Y2[k] = lin(Dp, zn[k]) + T2
            else:
                Y2[k] = lin(Pp, zn[k]) + lin(Qp, zn[kt])

    for i in range(3):
        for j in range(3):
            acc = Y2[3 * i + j]
            for t in range(3):
                acc = acc + Y2[3 * i + t] * Y2[3 * t + j]
            o_ref[3 * i + j] = acc


_dense = pl.pallas_call(
    _dense_body,
    grid=(NPAD // BN,),
    in_specs=[
        pl.BlockSpec((9, EMB, BN), lambda i: (0, 0, i)),
        pl.BlockSpec((2, RAD, BN), lambda i: (0, 0, i)),
        pl.BlockSpec((NW, BN), lambda i: (0, i)),
        pl.BlockSpec((4, EMB, EMB), lambda i: (0, 0, 0)),
        pl.BlockSpec((4, EMB, EMB), lambda i: (0, 0, 0)),
        pl.BlockSpec((3 * EMB, RAD), lambda i: (0, 0)),
        pl.BlockSpec((3 * EMB, 1), lambda i: (0, 0)),
    ],
    out_specs=pl.BlockSpec((9, EMB, BN), lambda i: (0, 0, i)),
    out_shape=jax.ShapeDtypeStruct((9, EMB, NPAD), jnp.float32),
)


def kernel(X, neighbour_index, positions, W_I_pre, W_A_pre, W_S_pre,
           W_I_post, W_A_post, W_S_post, lin_W, lin_b):
    ctr = neighbour_index[0]
    nbr = neighbour_index[1]
    px = jnp.asarray(positions[:, 0])
    py = jnp.asarray(positions[:, 1])
    pz = jnp.asarray(positions[:, 2])

    sc_gather, sc_scatter = _sc_kernels()
    q = sc_gather(px, py, pz, ctr, nbr)
    qp = jnp.pad(q, (0, EPAD_E - E))
    er, env = _rbf1(qp)
    gm_flat, g32_flat = sc_scatter(er, env, nbr)
    gm = gm_flat.reshape(2, RAD, NPAD)
    g32 = g32_flat.reshape(NW, NPAD)

    xt = jnp.pad(X.transpose(2, 3, 1, 0).reshape(9, EMB, N),
                 ((0, 0), (0, 0), (0, NPAD - N)))

    wpre = jnp.stack([0.5 * (W_A_pre + W_S_pre), 0.5 * (W_S_pre - W_A_pre),
                      W_S_pre, W_I_pre - W_S_pre])
    wpost = jnp.stack([0.5 * (W_A_post + W_S_post), 0.5 * (W_S_post - W_A_post),
                       W_S_post, W_I_post - W_S_post])

    out_t = _dense(xt, gm, g32, wpre, wpost, lin_W, lin_b[:, None])
    return out_t[:, :, :N].reshape(3, 3, EMB, N).transpose(3, 2, 0, 1)


# speedup vs baseline: 2.4844x; 1.2399x over previous
import functools

import numpy as np
import jax
import jax.numpy as jnp
from jax import lax
from jax.experimental import pallas as pl
from jax.experimental.pallas import tpu as pltpu
from jax.experimental.pallas import tpu_sc as plsc

N = 10000
E = 160000
EMB = 16
RAD = 32
CUTOFF = 5.0

NC = 2
NS = 16
NW = NC * NS
EPER = E // NW
GROUPS = EPER // 16
EPAD_T = (GROUPS + 1) * 16
NPAD = 10240
CH = 8000
NCHUNK = E // CH

_MEANS = np.linspace(np.exp(-CUTOFF), 1.0, RAD).astype(np.float32)
_BETA = float((2.0 / RAD * (1.0 - np.exp(-CUTOFF))) ** -2)


def _sc_gather_body(px_h, py_h, pz_h, ctr_h, nbr_h, q_h,
                    px_v, py_v, pz_v, ctr_v, nbr_v, q_v):
    cid = lax.axis_index("c")
    sid = lax.axis_index("s")
    wid = sid * NC + cid
    base = wid * EPER
    z16 = jnp.zeros((16,), jnp.int32)
    ctr_v[pl.ds(EPAD_T - 16, 16)] = z16
    nbr_v[pl.ds(EPAD_T - 16, 16)] = z16
    pltpu.sync_copy(px_h, px_v)
    pltpu.sync_copy(py_h, py_v)
    pltpu.sync_copy(pz_h, pz_v)
    pltpu.sync_copy(ctr_h.at[pl.ds(base, EPER)], ctr_v.at[pl.ds(0, EPER)])
    pltpu.sync_copy(nbr_h.at[pl.ds(base, EPER)], nbr_v.at[pl.ds(0, EPER)])

    def body(i, carry):
        ic = ctr_v[pl.ds(i * 16, 16)]
        inb = nbr_v[pl.ds(i * 16, 16)]
        dx = plsc.load_gather(px_v, [inb]) - plsc.load_gather(px_v, [ic])
        dy = plsc.load_gather(py_v, [inb]) - plsc.load_gather(py_v, [ic])
        dz = plsc.load_gather(pz_v, [inb]) - plsc.load_gather(pz_v, [ic])
        q_v[pl.ds(i * 16, 16)] = dx * dx + dy * dy + dz * dz
        return carry

    lax.fori_loop(0, GROUPS + 1, body, 0)
    pltpu.sync_copy(q_v.at[pl.ds(0, EPER)], q_h.at[pl.ds(base, EPER)])


UNR = 10
_M0 = float(_MEANS[0])
_DM = float((1.0 - np.exp(-CUTOFF)) / (RAD - 1))
_SQB = float(np.sqrt(_BETA))
KF = 4
EPART = E // KF
NCHUNK2 = EPART // CH


def _sc_scatter_body(er_h, env_h, nbr_h, gm_h, g32_h,
                     idx_a, era_v, eva_v, idx_b, erb_v, evb_v,
                     ei_v, ev_v, gs_v, g32_v,
                     sem_ia, sem_ea, sem_va, sem_ib, sem_eb, sem_vb):
    cid = lax.axis_index("c")
    sid = lax.axis_index("s")
    wid = sid * NC + cid
    k_ = lax.rem(wid, NW // KF)
    part = wid // (NW // KF)
    f0f = lax.convert_element_type(k_ * KF, jnp.float32)
    means = [(_M0 + (f0f + j) * _DM) * _SQB for j in range(KF)]

    zf = jnp.zeros((16,), jnp.float32)

    def zbody(i, carry):
        for u in range(8):
            for j in range(KF):
                gs_v[pl.ds(j * NPAD + i * 128 + u * 16, 16)] = zf
            g32_v[pl.ds(i * 128 + u * 16, 16)] = zf
        return carry

    lax.fori_loop(0, NPAD // 128, zbody, 0)

    bufs = ((idx_a, era_v, eva_v, sem_ia, sem_ea, sem_va),
            (idx_b, erb_v, evb_v, sem_ib, sem_eb, sem_vb))
    ebase0 = part * EPART

    def start(c, bi):
        ia, ea, va, si, se, sv = bufs[bi]
        d1 = pltpu.async_copy(nbr_h.at[pl.ds(ebase0 + c * CH, CH)], ia, si)
        d2 = pltpu.async_copy(er_h.at[pl.ds(ebase0 + c * CH, CH)], ea, se)
        d3 = pltpu.async_copy(env_h.at[pl.ds(ebase0 + c * CH, CH)], va, sv)
        return d1, d2, d3

    pend = start(0, 0)
    for c in range(NCHUNK2):
        bi = c & 1
        ia, ea, va, _, _, _ = bufs[bi]
        nxt = start(c + 1, 1 - bi) if c + 1 < NCHUNK2 else None
        pend[0].wait()
        pend[1].wait()
        pend[2].wait()

        def sbody(g, carry):
            for u in range(UNR):
                off = g * (16 * UNR) + u * 16
                iv = ia[pl.ds(off, 16)]
                erv = ea[pl.ds(off, 16)]
                env = va[pl.ds(off, 16)]
                for j in range(KF):
                    dj = erv - means[j]
                    ivj = iv + (j * NPAD)
                    plsc.addupdate_scatter(
                        gs_v, [ivj], jnp.exp(-(dj * dj)) * env)
            return carry

        lax.fori_loop(0, CH // (16 * UNR), sbody, 0)
        pend = nxt

    ebase = wid * EPER
    z16 = jnp.zeros((16,), jnp.int32)
    ei_v[pl.ds(EPAD_T - 16, 16)] = z16
    ev_v[pl.ds(EPAD_T - 16, 16)] = jnp.zeros((16,), jnp.float32)
    pltpu.sync_copy(nbr_h.at[pl.ds(ebase, EPER)], ei_v.at[pl.ds(0, EPER)])
    pltpu.sync_copy(env_h.at[pl.ds(ebase, EPER)], ev_v.at[pl.ds(0, EPER)])

    def ebody(g, carry):
        iv = ei_v[pl.ds(g * 16, 16)]
        vv = ev_v[pl.ds(g * 16, 16)]
        plsc.addupdate_scatter(g32_v, [iv], vv)
        return carry

    lax.fori_loop(0, GROUPS + 1, ebody, 0)

    pltpu.sync_copy(gs_v, gm_h.at[pl.ds(wid * (KF * NPAD), KF * NPAD)])
    pltpu.sync_copy(g32_v, g32_h.at[pl.ds(wid * NPAD, NPAD)])


@functools.cache
def _sc_kernels():
    mesh = plsc.VectorSubcoreMesh(
        core_axis_name="c", subcore_axis_name="s",
        num_cores=NC, num_subcores=NS)
    params = pltpu.CompilerParams(needs_layout_passes=False)
    sc_gather = pl.kernel(
        _sc_gather_body,
        out_type=jax.ShapeDtypeStruct((E,), jnp.float32),
        mesh=mesh,
        compiler_params=params,
        scratch_types=[
            pltpu.VMEM((N,), jnp.float32),
            pltpu.VMEM((N,), jnp.float32),
            pltpu.VMEM((N,), jnp.float32),
            pltpu.VMEM((EPAD_T,), jnp.int32),
            pltpu.VMEM((EPAD_T,), jnp.int32),
            pltpu.VMEM((EPAD_T,), jnp.float32),
        ],
    )
    sc_scatter = pl.kernel(
        _sc_scatter_body,
        out_type=(jax.ShapeDtypeStruct((KF * NW * NPAD,), jnp.float32),
                  jax.ShapeDtypeStruct((NW * NPAD,), jnp.float32)),
        mesh=mesh,
        compiler_params=params,
        scratch_types=[
            pltpu.VMEM((CH,), jnp.int32),
            pltpu.VMEM((CH,), jnp.float32),
            pltpu.VMEM((CH,), jnp.float32),
            pltpu.VMEM((CH,), jnp.int32),
            pltpu.VMEM((CH,), jnp.float32),
            pltpu.VMEM((CH,), jnp.float32),
            pltpu.VMEM((EPAD_T,), jnp.int32),
            pltpu.VMEM((EPAD_T,), jnp.float32),
            pltpu.VMEM((KF * NPAD,), jnp.float32),
            pltpu.VMEM((NPAD,), jnp.float32),
            pltpu.SemaphoreType.DMA,
            pltpu.SemaphoreType.DMA,
            pltpu.SemaphoreType.DMA,
            pltpu.SemaphoreType.DMA,
            pltpu.SemaphoreType.DMA,
            pltpu.SemaphoreType.DMA,
        ],
    )
    return sc_gather, sc_scatter


EPAD_E = 163840
BE = 16384


def _rbf1_body(q_ref, er_ref, env_ref):
    q = q_ref[...]
    r = jnp.sqrt(q + 1e-12)
    er_ref[...] = jnp.exp(-r) * np.float32(_SQB)
    env_ref[...] = jnp.where(
        r < CUTOFF, 0.5 * (jnp.cos(np.float32(np.pi) / CUTOFF * r) + 1.0), 0.0)


_rbf1 = pl.pallas_call(
    _rbf1_body,
    grid=(EPAD_E // BE,),
    in_specs=[pl.BlockSpec((BE,), lambda i: (i,))],
    out_specs=(pl.BlockSpec((BE,), lambda i: (i,)),
               pl.BlockSpec((BE,), lambda i: (i,))),
    out_shape=(jax.ShapeDtypeStruct((EPAD_E,), jnp.float32),
               jax.ShapeDtypeStruct((EPAD_E,), jnp.float32)),
)


BN = 1024


def _dense_body(xt_ref, gm_ref, g32_ref, wpre_ref, wpost_ref, wlin_ref,
                b_ref, o_ref):
    def lin(Wm, v):
        return jnp.dot(Wm, v, preferred_element_type=jnp.float32)

    x = [xt_ref[k] for k in range(9)]
    fro2 = x[0] * x[0]
    for k in range(1, 9):
        fro2 = fro2 + x[k] * x[k]
    inv = 1.0 / (fro2 + 1.0)
    xn = [x[k] * inv for k in range(9)]
    m = (xn[0] + xn[4] + xn[8]) / 3.0

    P, Q2, D_, T = (wpre_ref[0], wpre_ref[1], wpre_ref[2], wpre_ref[3])
    Tm = lin(T, m)
    Y = [None] * 9
    for i in range(3):
        for j in range(3):
            k, kt = 3 * i + j, 3 * j + i
            if i == j:
                Y[k] = lin(D_, xn[k]) + Tm
            else:
                Y[k] = lin(P, xn[k]) + lin(Q2, xn[kt])

    genv = jnp.sum(g32_ref[...], axis=0, keepdims=True)
    gsum = gm_ref[0] + gm_ref[1] + gm_ref[2] + gm_ref[3]
    C = lin(wlin_ref[...], gsum) + b_ref[...] * genv
    cI, cA, cS = C[0:EMB], C[EMB:2 * EMB], C[2 * EMB:3 * EMB]
    M = [None] * 9
    for i in range(3):
        for j in range(3):
            k, kt = 3 * i + j, 3 * j + i
            if i == j:
                M[k] = cI * m + cS * (xn[k] - m)
            else:
                M[k] = cA * (0.5 * (xn[k] - xn[kt])) + cS * (0.5 * (xn[k] + xn[kt]))

    Z = [None] * 9
    for i in range(3):
        for j in range(3):
            acc = None
            for t in range(3):
                term = Y[3 * i + t] * M[3 * t + j] + M[3 * i + t] * Y[3 * t + j]
                acc = term if acc is None else acc + term
            Z[3 * i + j] = acc

    n2 = None
    for k in range(9):
        zk1 = Z[k] + 1.0
        n2 = zk1 * zk1 if n2 is None else n2 + zk1 * zk1
    zn = [Z[k] / n2 for k in range(9)]
    m2 = (zn[0] + zn[4] + zn[8]) / 3.0

    Pp, Qp, Dp, Tp = (wpost_ref[0], wpost_ref[1], wpost_ref[2], wpost_ref[3])
    T2 = lin(Tp, m2)
    Y2 = [None] * 9
    for i in range(3):
        for j in range(3):
            k, kt = 3 * i + j, 3 * j + i
            if i == j:
                Y2[k] = lin(Dp, zn[k]) + T2
            else:
                Y2[k] = lin(Pp, zn[k]) + lin(Qp, zn[kt])

    for i in range(3):
        for j in range(3):
            acc = Y2[3 * i + j]
            for t in range(3):
                acc = acc + Y2[3 * i + t] * Y2[3 * t + j]
            o_ref[3 * i + j] = acc


_dense = pl.pallas_call(
    _dense_body,
    grid=(NPAD // BN,),
    in_specs=[
        pl.BlockSpec((9, EMB, BN), lambda i: (0, 0, i)),
        pl.BlockSpec((KF, RAD, BN), lambda i: (0, 0, i)),
        pl.BlockSpec((NW, BN), lambda i: (0, i)),
        pl.BlockSpec((4, EMB, EMB), lambda i: (0, 0, 0)),
        pl.BlockSpec((4, EMB, EMB), lambda i: (0, 0, 0)),
        pl.BlockSpec((3 * EMB, RAD), lambda i: (0, 0)),
        pl.BlockSpec((3 * EMB, 1), lambda i: (0, 0)),
    ],
    out_specs=pl.BlockSpec((9, EMB, BN), lambda i: (0, 0, i)),
    out_shape=jax.ShapeDtypeStruct((9, EMB, NPAD), jnp.float32),
)


def kernel(X, neighbour_index, positions, W_I_pre, W_A_pre, W_S_pre,
           W_I_post, W_A_post, W_S_post, lin_W, lin_b):
    ctr = neighbour_index[0]
    nbr = neighbour_index[1]
    px = jnp.asarray(positions[:, 0])
    py = jnp.asarray(positions[:, 1])
    pz = jnp.asarray(positions[:, 2])

    sc_gather, sc_scatter = _sc_kernels()
    q = sc_gather(px, py, pz, ctr, nbr)
    qp = jnp.pad(q, (0, EPAD_E - E))
    er, env = _rbf1(qp)
    gm_flat, g32_flat = sc_scatter(er, env, nbr)
    gm = gm_flat.reshape(KF, RAD, NPAD)
    g32 = g32_flat.reshape(NW, NPAD)

    xt = jnp.pad(X.transpose(2, 3, 1, 0).reshape(9, EMB, N),
                 ((0, 0), (0, 0), (0, NPAD - N)))

    wpre = jnp.stack([0.5 * (W_A_pre + W_S_pre), 0.5 * (W_S_pre - W_A_pre),
                      W_S_pre, W_I_pre - W_S_pre])
    wpost = jnp.stack([0.5 * (W_A_post + W_S_post), 0.5 * (W_S_post - W_A_post),
                       W_S_post, W_I_post - W_S_post])

    out_t = _dense(xt, gm, g32, wpre, wpost, lin_W, lin_b[:, None])
    return out_t[:, :, :N].reshape(3, 3, EMB, N).transpose(3, 2, 0, 1)


# speedup vs baseline: 2.6819x; 1.0795x over previous
import functools

import numpy as np
import jax
import jax.numpy as jnp
from jax import lax
from jax.experimental import pallas as pl
from jax.experimental.pallas import tpu as pltpu
from jax.experimental.pallas import tpu_sc as plsc

N = 10000
E = 160000
EMB = 16
RAD = 32
CUTOFF = 5.0

NC = 2
NS = 16
NW = NC * NS
EPER = E // NW
GROUPS = EPER // 16
EPAD_T = (GROUPS + 1) * 16
NPAD = 10240
CH = 4000

_MEANS = np.linspace(np.exp(-CUTOFF), 1.0, RAD).astype(np.float32)
_BETA = float((2.0 / RAD * (1.0 - np.exp(-CUTOFF))) ** -2)


def _sc_gather_body(px_h, py_h, pz_h, ctr_h, nbr_h, q_h,
                    px_v, py_v, pz_v, ctr_v, nbr_v, q_v):
    cid = lax.axis_index("c")
    sid = lax.axis_index("s")
    wid = sid * NC + cid
    base = wid * EPER
    z16 = jnp.zeros((16,), jnp.int32)
    ctr_v[pl.ds(EPAD_T - 16, 16)] = z16
    nbr_v[pl.ds(EPAD_T - 16, 16)] = z16
    pltpu.sync_copy(px_h, px_v)
    pltpu.sync_copy(py_h, py_v)
    pltpu.sync_copy(pz_h, pz_v)
    pltpu.sync_copy(ctr_h.at[pl.ds(base, EPER)], ctr_v.at[pl.ds(0, EPER)])
    pltpu.sync_copy(nbr_h.at[pl.ds(base, EPER)], nbr_v.at[pl.ds(0, EPER)])

    def body(i, carry):
        ic = ctr_v[pl.ds(i * 16, 16)]
        inb = nbr_v[pl.ds(i * 16, 16)]
        dx = plsc.load_gather(px_v, [inb]) - plsc.load_gather(px_v, [ic])
        dy = plsc.load_gather(py_v, [inb]) - plsc.load_gather(py_v, [ic])
        dz = plsc.load_gather(pz_v, [inb]) - plsc.load_gather(pz_v, [ic])
        q_v[pl.ds(i * 16, 16)] = dx * dx + dy * dy + dz * dz
        return carry

    lax.fori_loop(0, GROUPS + 1, body, 0)
    pltpu.sync_copy(q_v.at[pl.ds(0, EPER)], q_h.at[pl.ds(base, EPER)])


UNR = 10
_M0 = float(_MEANS[0])
_DM = float((1.0 - np.exp(-CUTOFF)) / (RAD - 1))
_SQB = float(np.sqrt(_BETA))
KF = 8
EPART = E // KF
NCHUNK2 = EPART // CH


def _sc_scatter_body(er_h, env_h, nbr_h, gm_h, g32_h,
                     idx_a, era_v, eva_v, idx_b, erb_v, evb_v,
                     ei_v, ev_v, gs_v, g32_v,
                     sem_ia, sem_ea, sem_va, sem_ib, sem_eb, sem_vb):
    cid = lax.axis_index("c")
    sid = lax.axis_index("s")
    wid = sid * NC + cid
    k_ = lax.rem(wid, NW // KF)
    part = wid // (NW // KF)
    f0f = lax.convert_element_type(k_ * KF, jnp.float32)
    means = [(_M0 + (f0f + j) * _DM) * _SQB for j in range(KF)]

    zf = jnp.zeros((16,), jnp.float32)

    def zbody(i, carry):
        for u in range(8):
            for j in range(KF):
                gs_v[pl.ds(j * NPAD + i * 128 + u * 16, 16)] = zf
            g32_v[pl.ds(i * 128 + u * 16, 16)] = zf
        return carry

    lax.fori_loop(0, NPAD // 128, zbody, 0)

    bufs = ((idx_a, era_v, eva_v, sem_ia, sem_ea, sem_va),
            (idx_b, erb_v, evb_v, sem_ib, sem_eb, sem_vb))
    ebase0 = part * EPART

    def start(c, bi):
        ia, ea, va, si, se, sv = bufs[bi]
        d1 = pltpu.async_copy(nbr_h.at[pl.ds(ebase0 + c * CH, CH)], ia, si)
        d2 = pltpu.async_copy(er_h.at[pl.ds(ebase0 + c * CH, CH)], ea, se)
        d3 = pltpu.async_copy(env_h.at[pl.ds(ebase0 + c * CH, CH)], va, sv)
        return d1, d2, d3

    pend = start(0, 0)
    for c in range(NCHUNK2):
        bi = c & 1
        ia, ea, va, _, _, _ = bufs[bi]
        nxt = start(c + 1, 1 - bi) if c + 1 < NCHUNK2 else None
        pend[0].wait()
        pend[1].wait()
        pend[2].wait()

        def sbody(g, carry):
            for u in range(UNR):
                off = g * (16 * UNR) + u * 16
                iv = ia[pl.ds(off, 16)]
                erv = ea[pl.ds(off, 16)]
                env = va[pl.ds(off, 16)]
                for j in range(KF):
                    dj = erv - means[j]
                    ivj = iv + (j * NPAD)
                    plsc.addupdate_scatter(
                        gs_v, [ivj], jnp.exp(-(dj * dj)) * env)
            return carry

        lax.fori_loop(0, CH // (16 * UNR), sbody, 0)
        pend = nxt

    ebase = wid * EPER
    z16 = jnp.zeros((16,), jnp.int32)
    ei_v[pl.ds(EPAD_T - 16, 16)] = z16
    ev_v[pl.ds(EPAD_T - 16, 16)] = jnp.zeros((16,), jnp.float32)
    pltpu.sync_copy(nbr_h.at[pl.ds(ebase, EPER)], ei_v.at[pl.ds(0, EPER)])
    pltpu.sync_copy(env_h.at[pl.ds(ebase, EPER)], ev_v.at[pl.ds(0, EPER)])

    def ebody(g, carry):
        iv = ei_v[pl.ds(g * 16, 16)]
        vv = ev_v[pl.ds(g * 16, 16)]
        plsc.addupdate_scatter(g32_v, [iv], vv)
        return carry

    lax.fori_loop(0, GROUPS + 1, ebody, 0)

    pltpu.sync_copy(gs_v, gm_h.at[pl.ds(wid * (KF * NPAD), KF * NPAD)])
    pltpu.sync_copy(g32_v, g32_h.at[pl.ds(wid * NPAD, NPAD)])


@functools.cache
def _sc_kernels():
    mesh = plsc.VectorSubcoreMesh(
        core_axis_name="c", subcore_axis_name="s",
        num_cores=NC, num_subcores=NS)
    params = pltpu.CompilerParams(needs_layout_passes=False)
    sc_gather = pl.kernel(
        _sc_gather_body,
        out_type=jax.ShapeDtypeStruct((E,), jnp.float32),
        mesh=mesh,
        compiler_params=params,
        scratch_types=[
            pltpu.VMEM((N,), jnp.float32),
            pltpu.VMEM((N,), jnp.float32),
            pltpu.VMEM((N,), jnp.float32),
            pltpu.VMEM((EPAD_T,), jnp.int32),
            pltpu.VMEM((EPAD_T,), jnp.int32),
            pltpu.VMEM((EPAD_T,), jnp.float32),
        ],
    )
    sc_scatter = pl.kernel(
        _sc_scatter_body,
        out_type=(jax.ShapeDtypeStruct((KF * NW * NPAD,), jnp.float32),
                  jax.ShapeDtypeStruct((NW * NPAD,), jnp.float32)),
        mesh=mesh,
        compiler_params=params,
        scratch_types=[
            pltpu.VMEM((CH,), jnp.int32),
            pltpu.VMEM((CH,), jnp.float32),
            pltpu.VMEM((CH,), jnp.float32),
            pltpu.VMEM((CH,), jnp.int32),
            pltpu.VMEM((CH,), jnp.float32),
            pltpu.VMEM((CH,), jnp.float32),
            pltpu.VMEM((EPAD_T,), jnp.int32),
            pltpu.VMEM((EPAD_T,), jnp.float32),
            pltpu.VMEM((KF * NPAD,), jnp.float32),
            pltpu.VMEM((NPAD,), jnp.float32),
            pltpu.SemaphoreType.DMA,
            pltpu.SemaphoreType.DMA,
            pltpu.SemaphoreType.DMA,
            pltpu.SemaphoreType.DMA,
            pltpu.SemaphoreType.DMA,
            pltpu.SemaphoreType.DMA,
        ],
    )
    return sc_gather, sc_scatter


EPAD_E = 163840
BE = 16384


def _rbf1_body(q_ref, er_ref, env_ref):
    q = q_ref[...]
    r = jnp.sqrt(q + 1e-12)
    er_ref[...] = jnp.exp(-r) * np.float32(_SQB)
    env_ref[...] = jnp.where(
        r < CUTOFF, 0.5 * (jnp.cos(np.float32(np.pi) / CUTOFF * r) + 1.0), 0.0)


_rbf1 = pl.pallas_call(
    _rbf1_body,
    grid=(EPAD_E // BE,),
    in_specs=[pl.BlockSpec((BE,), lambda i: (i,))],
    out_specs=(pl.BlockSpec((BE,), lambda i: (i,)),
               pl.BlockSpec((BE,), lambda i: (i,))),
    out_shape=(jax.ShapeDtypeStruct((EPAD_E,), jnp.float32),
               jax.ShapeDtypeStruct((EPAD_E,), jnp.float32)),
)


BN = 1024


def _dense_body(xt_ref, gm_ref, g32_ref, wpre_ref, wpost_ref, wlin_ref,
                b_ref, o_ref):
    def lin(Wm, v):
        return jnp.dot(Wm, v, preferred_element_type=jnp.float32)

    x = [xt_ref[k] for k in range(9)]
    fro2 = x[0] * x[0]
    for k in range(1, 9):
        fro2 = fro2 + x[k] * x[k]
    inv = 1.0 / (fro2 + 1.0)
    xn = [x[k] * inv for k in range(9)]
    m = (xn[0] + xn[4] + xn[8]) / 3.0

    P, Q2, D_, T = (wpre_ref[0], wpre_ref[1], wpre_ref[2], wpre_ref[3])
    Tm = lin(T, m)
    Y = [None] * 9
    for i in range(3):
        for j in range(3):
            k, kt = 3 * i + j, 3 * j + i
            if i == j:
                Y[k] = lin(D_, xn[k]) + Tm
            else:
                Y[k] = lin(P, xn[k]) + lin(Q2, xn[kt])

    genv = jnp.sum(g32_ref[...], axis=0, keepdims=True)
    gsum = gm_ref[0]
    for p in range(1, KF):
        gsum = gsum + gm_ref[p]
    C = lin(wlin_ref[...], gsum) + b_ref[...] * genv
    cI, cA, cS = C[0:EMB], C[EMB:2 * EMB], C[2 * EMB:3 * EMB]
    M = [None] * 9
    for i in range(3):
        for j in range(3):
            k, kt = 3 * i + j, 3 * j + i
            if i == j:
                M[k] = cI * m + cS * (xn[k] - m)
            else:
                M[k] = cA * (0.5 * (xn[k] - xn[kt])) + cS * (0.5 * (xn[k] + xn[kt]))

    Z = [None] * 9
    for i in range(3):
        for j in range(3):
            acc = None
            for t in range(3):
                term = Y[3 * i + t] * M[3 * t + j] + M[3 * i + t] * Y[3 * t + j]
                acc = term if acc is None else acc + term
            Z[3 * i + j] = acc

    n2 = None
    for k in range(9):
        zk1 = Z[k] + 1.0
        n2 = zk1 * zk1 if n2 is None else n2 + zk1 * zk1
    zn = [Z[k] / n2 for k in range(9)]
    m2 = (zn[0] + zn[4] + zn[8]) / 3.0

    Pp, Qp, Dp, Tp = (wpost_ref[0], wpost_ref[1], wpost_ref[2], wpost_ref[3])
    T2 = lin(Tp, m2)
    Y2 = [None] * 9
    for i in range(3):
        for j in range(3):
            k, kt = 3 * i + j, 3 * j + i
            if i == j:
                Y2[k] = lin(Dp, zn[k]) + T2
            else:
                Y2[k] = lin(Pp, zn[k]) + lin(Qp, zn[kt])

    for i in range(3):
        for j in range(3):
            acc = Y2[3 * i + j]
            for t in range(3):
                acc = acc + Y2[3 * i + t] * Y2[3 * t + j]
            o_ref[3 * i + j] = acc


_dense = pl.pallas_call(
    _dense_body,
    grid=(NPAD // BN,),
    in_specs=[
        pl.BlockSpec((9, EMB, BN), lambda i: (0, 0, i)),
        pl.BlockSpec((KF, RAD, BN), lambda i: (0, 0, i)),
        pl.BlockSpec((NW, BN), lambda i: (0, i)),
        pl.BlockSpec((4, EMB, EMB), lambda i: (0, 0, 0)),
        pl.BlockSpec((4, EMB, EMB), lambda i: (0, 0, 0)),
        pl.BlockSpec((3 * EMB, RAD), lambda i: (0, 0)),
        pl.BlockSpec((3 * EMB, 1), lambda i: (0, 0)),
    ],
    out_specs=pl.BlockSpec((9, EMB, BN), lambda i: (0, 0, i)),
    out_shape=jax.ShapeDtypeStruct((9, EMB, NPAD), jnp.float32),
)


def kernel(X, neighbour_index, positions, W_I_pre, W_A_pre, W_S_pre,
           W_I_post, W_A_post, W_S_post, lin_W, lin_b):
    ctr = neighbour_index[0]
    nbr = neighbour_index[1]
    px = jnp.asarray(positions[:, 0])
    py = jnp.asarray(positions[:, 1])
    pz = jnp.asarray(positions[:, 2])

    sc_gather, sc_scatter = _sc_kernels()
    q = sc_gather(px, py, pz, ctr, nbr)
    qp = jnp.pad(q, (0, EPAD_E - E))
    er, env = _rbf1(qp)
    gm_flat, g32_flat = sc_scatter(er, env, nbr)
    gm = gm_flat.reshape(KF, RAD, NPAD)
    g32 = g32_flat.reshape(NW, NPAD)

    xt = jnp.pad(X.transpose(2, 3, 1, 0).reshape(9, EMB, N),
                 ((0, 0), (0, 0), (0, NPAD - N)))

    wpre = jnp.stack([0.5 * (W_A_pre + W_S_pre), 0.5 * (W_S_pre - W_A_pre),
                      W_S_pre, W_I_pre - W_S_pre])
    wpost = jnp.stack([0.5 * (W_A_post + W_S_post), 0.5 * (W_S_post - W_A_post),
                       W_S_post, W_I_post - W_S_post])

    out_t = _dense(xt, gm, g32, wpre, wpost, lin_W, lin_b[:, None])
    return out_t[:, :, :N].reshape(3, 3, EMB, N).transpose(3, 2, 0, 1)
